# baseline (speedup 1.0000x reference)
import functools

import jax
import jax.numpy as jnp
from jax import lax
from jax.experimental import pallas as pl
from jax.experimental.pallas import tpu as pltpu
from jax.experimental.pallas import tpu_sc as plsc

N = 16384
M = 16384
K = 16
B = 64
NW = 32
RPW = N // NW
CR = 32
NCH = RPW // CR
NI = CR * K
GB = 4
LB = 16


def _body(xT_hbm, w_hbm, bias_hbm, idx_hbm, out_hbm,
          idx_v, g_v, w_v, b_v, o_v, sem_ga, sem_gb, sem_o):
    wid = lax.axis_index("s") * 2 + lax.axis_index("c")
    row0 = wid * RPW

    pltpu.sync_copy(idx_hbm.at[pl.ds(wid * NCH, NCH)], idx_v)
    pltpu.sync_copy(w_hbm.at[pl.ds(row0 * K, RPW * K)], w_v)
    pltpu.sync_copy(bias_hbm.at[pl.ds(row0, RPW)], b_v)

    def gathers(c, p):
        for j in range(GB):
            sem = sem_ga if j < 2 else sem_gb
            pltpu.async_copy(xT_hbm.at[idx_v.at[c, j]], g_v.at[p, j],
                             sem.at[p])

    def drain(sem, p, nblocks):
        for _ in range(nblocks):
            pltpu.make_async_copy(xT_hbm.at[idx_v.at[0, 0]], g_v.at[p, 0],
                                  sem.at[p]).wait()

    gathers(0, 0)

    def chunk(c, _):
        p = lax.rem(c, 2)

        @pl.when(c + 1 < NCH)
        def _():
            gathers(c + 1, 1 - p)

        @pl.when(c >= 2)
        def _():
            pltpu.make_async_copy(o_v.at[p], out_hbm.at[pl.ds(row0, CR)],
                                  sem_o.at[p]).wait()

        def row_group(g, _):
            @pl.when(g == 0)
            def _():
                drain(sem_ga, p, 2)

            @pl.when(g == 1)
            def _():
                drain(sem_gb, p, 2)

            bgrp = b_v[pl.ds(c * CR + g * LB, LB)]
            for l in range(LB):
                wrow = w_v[pl.ds(c * NI + g * 256 + l * K, K)]
                blk = g * 2 + (l // 8)
                r0 = (l % 8) * K
                accs = [jnp.full((LB,), bgrp[l], dtype=jnp.float32)
                        for _ in range(B // LB)]
                for k in range(K):
                    wv = jnp.full((LB,), wrow[k], dtype=jnp.float32)
                    for h in range(2):
                        packed = g_v[p, blk, r0 + k, pl.ds(h * 32, 32)]
                        a, b = plsc.unpack(packed,
                                           format=plsc.PackFormat.INTERLEAVED)
                        accs[2 * h] = accs[2 * h] + wv * a
                        accs[2 * h + 1] = accs[2 * h + 1] + wv * b
                for j in range(B // LB):
                    o_v[p, g * LB + l, pl.ds(j * LB, LB)] = accs[j]
            return ()

        lax.fori_loop(0, CR // LB, row_group, (), unroll=False)
        pltpu.async_copy(o_v.at[p],
                         out_hbm.at[pl.ds(row0 + c * CR, CR)], sem_o.at[p])
        return ()

    lax.fori_loop(0, NCH, chunk, (), unroll=False)
    for p in range(2):
        pltpu.make_async_copy(o_v.at[p], out_hbm.at[pl.ds(row0, CR)],
                              sem_o.at[p]).wait()


@jax.jit
def _spmm(xT, W_val, bias, idx3):
    mesh = plsc.VectorSubcoreMesh(core_axis_name="c", subcore_axis_name="s")
    f = pl.kernel(
        _body,
        out_type=jax.ShapeDtypeStruct((N, B), jnp.float32),
        mesh=mesh,
        scratch_types=[
            pltpu.VMEM((NCH, GB, 128), jnp.int32),
            pltpu.VMEM((2, GB, 128, B), jnp.bfloat16),
            pltpu.VMEM((RPW * K,), jnp.float32),
            pltpu.VMEM((RPW,), jnp.float32),
            pltpu.VMEM((2, CR, B), jnp.float32),
            pltpu.SemaphoreType.DMA((2,)),
            pltpu.SemaphoreType.DMA((2,)),
            pltpu.SemaphoreType.DMA((2,)),
        ],
        compiler_params=pltpu.CompilerParams(use_tc_tiling_on_sc=False,
                                             needs_layout_passes=False),
    )
    return f(xT, W_val, bias, idx3)


def kernel(input, W_val, bias, indices, rows):
    x2 = input.reshape(-1, input.shape[-1])
    xT = x2.T.astype(jnp.bfloat16)
    xTs = xT.reshape(M, 2, 2, 16).transpose(0, 1, 3, 2).reshape(M, B)
    idx3 = indices.reshape(-1, GB, 128)
    yT = _spmm(xTs, W_val, bias, idx3)
    return yT.T.reshape(input.shape[:-1] + (N,))


# speedup vs baseline: 1.0047x; 1.0047x over previous
import functools

import jax
import jax.numpy as jnp
import numpy as np
from jax import lax
from jax.experimental import pallas as pl
from jax.experimental.pallas import tpu as pltpu
from jax.experimental.pallas import tpu_sc as plsc

N = 16384
M = 16384
K = 16
B = 64
NW = 32
RPW = N // NW
CR = 32
NCH = RPW // CR
NI = CR * K
GB = 4
LB = 16


def _body(xT_hbm, w_hbm, bias_hbm, idx_hbm, out_hbm,
          idx_v, g_v, w_v, b_v, o_v, sem_ga, sem_gb, sem_o):
    wid = lax.axis_index("s") * 2 + lax.axis_index("c")
    row0 = wid * RPW

    pltpu.sync_copy(idx_hbm.at[pl.ds(wid * NCH, NCH)], idx_v)
    pltpu.sync_copy(w_hbm.at[pl.ds(row0 * K, RPW * K)], w_v)
    pltpu.sync_copy(bias_hbm.at[pl.ds(row0, RPW)], b_v)

    def gathers(c, p):
        for j in range(GB):
            sem = sem_ga if j < 2 else sem_gb
            pltpu.async_copy(xT_hbm.at[idx_v.at[c, j]], g_v.at[p, j],
                             sem.at[p])

    def drain(sem, p, nblocks):
        for _ in range(nblocks):
            pltpu.make_async_copy(xT_hbm.at[idx_v.at[0, 0]], g_v.at[p, 0],
                                  sem.at[p]).wait()

    gathers(0, 0)

    def chunk(c, _):
        p = lax.rem(c, 2)

        @pl.when(c + 1 < NCH)
        def _():
            gathers(c + 1, 1 - p)

        @pl.when(c >= 2)
        def _():
            pltpu.make_async_copy(o_v.at[p], out_hbm.at[pl.ds(row0, CR)],
                                  sem_o.at[p]).wait()

        def row_group(g, _):
            @pl.when(g == 0)
            def _():
                drain(sem_ga, p, 2)

            @pl.when(g == 1)
            def _():
                drain(sem_gb, p, 2)

            bgrp = b_v[pl.ds(c * CR + g * LB, LB)]
            for l in range(LB):
                wrow = w_v[pl.ds(c * NI + g * 256 + l * K, K)]
                blk = g * 2 + (l // 8)
                r0 = (l % 8) * K
                accs = [jnp.full((LB,), bgrp[l], dtype=jnp.float32)
                        for _ in range(B // LB)]
                for k in range(K):
                    wv = jnp.full((LB,), wrow[k], dtype=jnp.float32)
                    for h in range(2):
                        packed = g_v[p, blk, r0 + k, pl.ds(h * 32, 32)]
                        a, b = plsc.unpack(packed,
                                           format=plsc.PackFormat.INTERLEAVED)
                        accs[2 * h] = accs[2 * h] + wv * a
                        accs[2 * h + 1] = accs[2 * h + 1] + wv * b
                for j in range(B // LB):
                    o_v[p, g * LB + l, pl.ds(j * LB, LB)] = accs[j]
            return ()

        lax.fori_loop(0, CR // LB, row_group, (), unroll=False)
        pltpu.async_copy(o_v.at[p],
                         out_hbm.at[pl.ds(row0 + c * CR, CR)], sem_o.at[p])
        return ()

    lax.fori_loop(0, NCH, chunk, (), unroll=False)
    for p in range(2):
        pltpu.make_async_copy(o_v.at[p], out_hbm.at[pl.ds(row0, CR)],
                              sem_o.at[p]).wait()


@jax.jit
def _spmm(xT, W_val, bias, idx3):
    mesh = plsc.VectorSubcoreMesh(core_axis_name="c", subcore_axis_name="s")
    f = pl.kernel(
        _body,
        out_type=jax.ShapeDtypeStruct((N, B), jnp.float32),
        mesh=mesh,
        scratch_types=[
            pltpu.VMEM((NCH, GB, 128), jnp.int32),
            pltpu.VMEM((2, GB, 128, B), jnp.bfloat16),
            pltpu.VMEM((RPW * K,), jnp.float32),
            pltpu.VMEM((RPW,), jnp.float32),
            pltpu.VMEM((2, CR, B), jnp.float32),
            pltpu.SemaphoreType.DMA((2,)),
            pltpu.SemaphoreType.DMA((2,)),
            pltpu.SemaphoreType.DMA((2,)),
        ],
        compiler_params=pltpu.CompilerParams(use_tc_tiling_on_sc=False,
                                             needs_layout_passes=False),
    )
    return f(xT, W_val, bias, idx3)


def kernel(input, W_val, bias, indices, rows):
    x2 = input.reshape(-1, input.shape[-1])
    perm = np.arange(B).reshape(2, 2, 16).transpose(0, 2, 1).reshape(B)
    xTs = x2[perm, :].astype(jnp.bfloat16).T
    idx3 = indices.reshape(-1, GB, 128)
    yT = _spmm(xTs, W_val, bias, idx3)
    return yT.T.reshape(input.shape[:-1] + (N,))


# speedup vs baseline: 1.0946x; 1.0895x over previous
import functools

import jax
import jax.numpy as jnp
import numpy as np
from jax import lax
from jax.experimental import pallas as pl
from jax.experimental.pallas import tpu as pltpu
from jax.experimental.pallas import tpu_sc as plsc

N = 16384
M = 16384
K = 16
B = 64
NW = 32
RPW = N // NW
CR = 32
NCH = RPW // CR
NI = CR * K
GB = 4
LB = 16
WBK = 4


def _body(xT_hbm, w_hbm, bias_hbm, idx_hbm, out_hbm,
          idx_v, g_v, w_v, b_v, o_t, sem_ga, sem_gb, sem_o):
    wid = lax.axis_index("s") * 2 + lax.axis_index("c")
    row0 = wid * RPW
    bidxs = [lax.iota(jnp.int32, LB) + (j * LB) for j in range(B // LB)]

    pltpu.sync_copy(idx_hbm.at[pl.ds(wid * NCH, NCH)], idx_v)
    pltpu.sync_copy(w_hbm.at[pl.ds(row0 * K, RPW * K)], w_v)
    pltpu.sync_copy(bias_hbm.at[pl.ds(row0, RPW)], b_v)

    def gathers(c, p):
        for j in range(GB):
            sem = sem_ga if j < 2 else sem_gb
            pltpu.async_copy(xT_hbm.at[idx_v.at[c, j]], g_v.at[p, j],
                             sem.at[p])

    def drain(sem, p, nblocks):
        for _ in range(nblocks):
            pltpu.make_async_copy(xT_hbm.at[idx_v.at[0, 0]], g_v.at[p, 0],
                                  sem.at[p]).wait()

    gathers(0, 0)

    def chunk(c, _):
        p = lax.rem(c, 2)

        @pl.when(c + 1 < NCH)
        def _():
            gathers(c + 1, 1 - p)

        def row_group(g, _):
            @pl.when(g == 0)
            def _():
                drain(sem_ga, p, 2)

            @pl.when(g == 1)
            def _():
                drain(sem_gb, p, 2)

            bgrp = b_v[pl.ds(c * CR + g * LB, LB)]
            for l in range(LB):
                wrow = w_v[pl.ds(c * NI + g * 256 + l * K, K)]
                blk = g * 2 + (l // 8)
                r0 = (l % 8) * K
                accs = [jnp.full((LB,), bgrp[l], dtype=jnp.float32)
                        for _ in range(B // LB)]
                for k in range(K):
                    wv = jnp.full((LB,), wrow[k], dtype=jnp.float32)
                    for h in range(2):
                        packed = g_v[p, blk, r0 + k, pl.ds(h * 32, 32)]
                        a, b = plsc.unpack(packed,
                                           format=plsc.PackFormat.INTERLEAVED)
                        accs[2 * h] = accs[2 * h] + wv * a
                        accs[2 * h + 1] = accs[2 * h + 1] + wv * b
                rvec = jnp.full((LB,), c * CR + g * LB + l, dtype=jnp.int32)
                for j in range(B // LB):
                    plsc.store_scatter(o_t, [bidxs[j], rvec], accs[j])
            return ()

        lax.fori_loop(0, CR // LB, row_group, (), unroll=False)

        @pl.when(lax.rem(c, NCH // WBK) == (NCH // WBK - 1))
        def _():
            q = c // (NCH // WBK)
            cols = RPW // WBK
            pltpu.async_copy(
                o_t.at[:, pl.ds(q * cols, cols)],
                out_hbm.at[:, pl.ds(row0 + q * cols, cols)], sem_o)
        return ()

    lax.fori_loop(0, NCH, chunk, (), unroll=False)
    for q in range(WBK):
        cols = RPW // WBK
        pltpu.make_async_copy(o_t.at[:, pl.ds(q * cols, cols)],
                              out_hbm.at[:, pl.ds(row0, cols)], sem_o).wait()


@jax.jit
def _spmm(xT, W_val, bias, idx3):
    mesh = plsc.VectorSubcoreMesh(core_axis_name="c", subcore_axis_name="s")
    f = pl.kernel(
        _body,
        out_type=jax.ShapeDtypeStruct((B, N), jnp.float32),
        mesh=mesh,
        scratch_types=[
            pltpu.VMEM((NCH, GB, 128), jnp.int32),
            pltpu.VMEM((2, GB, 128, B), jnp.bfloat16),
            pltpu.VMEM((RPW * K,), jnp.float32),
            pltpu.VMEM((RPW,), jnp.float32),
            pltpu.VMEM((B, RPW + 1), jnp.float32),
            pltpu.SemaphoreType.DMA((2,)),
            pltpu.SemaphoreType.DMA((2,)),
            pltpu.SemaphoreType.DMA,
        ],
        compiler_params=pltpu.CompilerParams(use_tc_tiling_on_sc=False,
                                             needs_layout_passes=False),
    )
    return f(xT, W_val, bias, idx3)


def kernel(input, W_val, bias, indices, rows):
    x2 = input.reshape(-1, input.shape[-1])
    perm = np.arange(B).reshape(2, 2, 16).transpose(0, 2, 1).reshape(B)
    xTs = x2[perm, :].astype(jnp.bfloat16).T
    idx3 = indices.reshape(-1, GB, 128)
    y = _spmm(xTs, W_val, bias, idx3)
    return y.reshape(input.shape[:-1] + (N,))


# speedup vs baseline: 1.1570x; 1.0570x over previous
import functools

import jax
import jax.numpy as jnp
import numpy as np
from jax import lax
from jax.experimental import pallas as pl
from jax.experimental.pallas import tpu as pltpu
from jax.experimental.pallas import tpu_sc as plsc

N = 16384
M = 16384
K = 16
B = 64
NW = 32
RPW = N // NW
CR = 32
NCH = RPW // CR
NI = CR * K
GB = 4
LB = 16
WBK = 4


def _body(xT_hbm, w_hbm, bias_hbm, idx_hbm, out_hbm,
          idx_v, g_v, w_v, b_v, o_t, sem_ga, sem_gb, sem_o):
    wid = lax.axis_index("s") * 2 + lax.axis_index("c")
    row0 = wid * RPW
    iot2 = lax.iota(jnp.int32, LB) * 2
    bidxs = [iot2 + (32 * h + e) for h in range(2) for e in range(2)]

    pltpu.sync_copy(idx_hbm.at[pl.ds(wid * NCH, NCH)], idx_v)
    pltpu.sync_copy(w_hbm.at[pl.ds(row0 * K, RPW * K)], w_v)
    pltpu.sync_copy(bias_hbm.at[pl.ds(row0, RPW)], b_v)

    def gathers(c, p):
        for j in range(GB):
            sem = sem_ga if j < 2 else sem_gb
            pltpu.async_copy(xT_hbm.at[idx_v.at[c, j]], g_v.at[p, j],
                             sem.at[p])

    def drain(sem, p, nblocks):
        for _ in range(nblocks):
            pltpu.make_async_copy(xT_hbm.at[idx_v.at[0, 0]], g_v.at[p, 0],
                                  sem.at[p]).wait()

    gathers(0, 0)

    def chunk(c, _):
        p = lax.rem(c, 2)

        @pl.when(c + 1 < NCH)
        def _():
            gathers(c + 1, 1 - p)

        def row_group(g, _):
            @pl.when(g == 0)
            def _():
                drain(sem_ga, p, 2)

            @pl.when(g == 1)
            def _():
                drain(sem_gb, p, 2)

            bgrp = b_v[pl.ds(c * CR + g * LB, LB)]
            for l in range(LB):
                wrow = w_v[pl.ds(c * NI + g * 256 + l * K, K)]
                blk = g * 2 + (l // 8)
                r0 = (l % 8) * K
                accs = [jnp.full((LB,), bgrp[l], dtype=jnp.float32)
                        for _ in range(B // LB)]
                for k in range(K):
                    wv = jnp.full((LB,), wrow[k], dtype=jnp.float32)
                    for h in range(2):
                        packed = g_v[p, blk, r0 + k, pl.ds(h * 32, 32)]
                        a, b = plsc.unpack(packed,
                                           format=plsc.PackFormat.INTERLEAVED)
                        accs[2 * h] = accs[2 * h] + wv * a
                        accs[2 * h + 1] = accs[2 * h + 1] + wv * b
                rvec = jnp.full((LB,), c * CR + g * LB + l, dtype=jnp.int32)
                for j in range(B // LB):
                    plsc.store_scatter(o_t, [bidxs[j], rvec], accs[j])
            return ()

        lax.fori_loop(0, CR // LB, row_group, (), unroll=False)

        @pl.when(lax.rem(c, NCH // WBK) == (NCH // WBK - 1))
        def _():
            q = c // (NCH // WBK)
            cols = RPW // WBK
            pltpu.async_copy(
                o_t.at[:, pl.ds(q * cols, cols)],
                out_hbm.at[:, pl.ds(row0 + q * cols, cols)], sem_o)
        return ()

    lax.fori_loop(0, NCH, chunk, (), unroll=False)
    for q in range(WBK):
        cols = RPW // WBK
        pltpu.make_async_copy(o_t.at[:, pl.ds(q * cols, cols)],
                              out_hbm.at[:, pl.ds(row0, cols)], sem_o).wait()


@jax.jit
def _spmm(xT, W_val, bias, idx3):
    mesh = plsc.VectorSubcoreMesh(core_axis_name="c", subcore_axis_name="s")
    f = pl.kernel(
        _body,
        out_type=jax.ShapeDtypeStruct((B, N), jnp.float32),
        mesh=mesh,
        scratch_types=[
            pltpu.VMEM((NCH, GB, 128), jnp.int32),
            pltpu.VMEM((2, GB, 128, B), jnp.bfloat16),
            pltpu.VMEM((RPW * K,), jnp.float32),
            pltpu.VMEM((RPW,), jnp.float32),
            pltpu.VMEM((B, RPW + 1), jnp.float32),
            pltpu.SemaphoreType.DMA((2,)),
            pltpu.SemaphoreType.DMA((2,)),
            pltpu.SemaphoreType.DMA,
        ],
        compiler_params=pltpu.CompilerParams(use_tc_tiling_on_sc=False,
                                             needs_layout_passes=False),
    )
    return f(xT, W_val, bias, idx3)


def kernel(input, W_val, bias, indices, rows):
    x2 = input.reshape(-1, input.shape[-1])
    xTs = x2.astype(jnp.bfloat16).T
    idx3 = indices.reshape(-1, GB, 128)
    y = _spmm(xTs, W_val, bias, idx3)
    return y.reshape(input.shape[:-1] + (N,))


# speedup vs baseline: 1.2916x; 1.1163x over previous
import functools

import jax
import jax.numpy as jnp
import numpy as np
from jax import lax
from jax.experimental import pallas as pl
from jax.experimental.pallas import tpu as pltpu
from jax.experimental.pallas import tpu_sc as plsc

N = 16384
M = 16384
K = 16
B = 64
NW = 32
RPW = N // NW
CR = 32
NCH = RPW // CR
NI = CR * K
GB = 4
LB = 16
WBK = 4


def _body(xT_hbm, w_hbm, bias_hbm, idx_hbm, out_hbm,
          idx_v, g_v, w_v, b_v, o_t, sem_ga, sem_gb, sem_o):
    wid = lax.axis_index("s") * 2 + lax.axis_index("c")
    row0 = wid * RPW
    iot2 = lax.iota(jnp.int32, LB) * 2
    bidxs = [iot2 + (32 * h + e) for h in range(2) for e in range(2)]

    pltpu.sync_copy(idx_hbm.at[pl.ds(wid * NCH, NCH)], idx_v)
    pltpu.sync_copy(w_hbm.at[pl.ds(row0 * K, RPW * K)], w_v)
    pltpu.sync_copy(bias_hbm.at[pl.ds(row0, RPW)], b_v)

    def gathers(c, p):
        for j in range(GB):
            sem = sem_ga if j < 2 else sem_gb
            pltpu.async_copy(xT_hbm.at[idx_v.at[c, j]], g_v.at[p, j],
                             sem.at[p])

    def drain(sem, p, nblocks):
        for _ in range(nblocks):
            pltpu.make_async_copy(xT_hbm.at[idx_v.at[0, 0]], g_v.at[p, 0],
                                  sem.at[p]).wait()

    gathers(0, 0)

    def chunk(c, _):
        p = lax.rem(c, 2)

        @pl.when(c + 1 < NCH)
        def _():
            gathers(c + 1, 1 - p)

        def row_group(g, _):
            @pl.when(g == 0)
            def _():
                drain(sem_ga, p, 2)

            @pl.when(g == 1)
            def _():
                drain(sem_gb, p, 2)

            bgrp = b_v[pl.ds(c * CR + g * LB, LB)]
            for l in range(LB):
                wrow = w_v[pl.ds(c * NI + g * 256 + l * K, K)]
                blk = g * 2 + (l // 8)
                r0 = (l % 8) * K
                accs = [jnp.zeros((2 * LB,), dtype=jnp.bfloat16)
                        for _ in range(2)]
                for k in range(K):
                    wf = jnp.full((LB,), wrow[k], dtype=jnp.float32)
                    wv = plsc.pack(wf, wf,
                                   format=plsc.PackFormat.INTERLEAVED)
                    for h in range(2):
                        packed = g_v[p, blk, r0 + k, pl.ds(h * 32, 32)]
                        accs[h] = accs[h] + wv * packed
                bval = bgrp[l]
                rvec = jnp.full((LB,), c * CR + g * LB + l, dtype=jnp.int32)
                for h in range(2):
                    a, b = plsc.unpack(accs[h],
                                       format=plsc.PackFormat.INTERLEAVED)
                    plsc.store_scatter(o_t, [bidxs[2 * h], rvec], a + bval)
                    plsc.store_scatter(o_t, [bidxs[2 * h + 1], rvec],
                                       b + bval)
            return ()

        lax.fori_loop(0, CR // LB, row_group, (), unroll=False)

        @pl.when(lax.rem(c, NCH // WBK) == (NCH // WBK - 1))
        def _():
            q = c // (NCH // WBK)
            cols = RPW // WBK
            pltpu.async_copy(
                o_t.at[:, pl.ds(q * cols, cols)],
                out_hbm.at[:, pl.ds(row0 + q * cols, cols)], sem_o)
        return ()

    lax.fori_loop(0, NCH, chunk, (), unroll=False)
    for q in range(WBK):
        cols = RPW // WBK
        pltpu.make_async_copy(o_t.at[:, pl.ds(q * cols, cols)],
                              out_hbm.at[:, pl.ds(row0, cols)], sem_o).wait()


@jax.jit
def _spmm(xT, W_val, bias, idx3):
    mesh = plsc.VectorSubcoreMesh(core_axis_name="c", subcore_axis_name="s")
    f = pl.kernel(
        _body,
        out_type=jax.ShapeDtypeStruct((B, N), jnp.float32),
        mesh=mesh,
        scratch_types=[
            pltpu.VMEM((NCH, GB, 128), jnp.int32),
            pltpu.VMEM((2, GB, 128, B), jnp.bfloat16),
            pltpu.VMEM((RPW * K,), jnp.float32),
            pltpu.VMEM((RPW,), jnp.float32),
            pltpu.VMEM((B, RPW + 1), jnp.float32),
            pltpu.SemaphoreType.DMA((2,)),
            pltpu.SemaphoreType.DMA((2,)),
            pltpu.SemaphoreType.DMA,
        ],
        compiler_params=pltpu.CompilerParams(use_tc_tiling_on_sc=False,
                                             needs_layout_passes=False),
    )
    return f(xT, W_val, bias, idx3)


def kernel(input, W_val, bias, indices, rows):
    x2 = input.reshape(-1, input.shape[-1])
    xTs = x2.astype(jnp.bfloat16).T
    idx3 = indices.reshape(-1, GB, 128)
    y = _spmm(xTs, W_val, bias, idx3)
    return y.reshape(input.shape[:-1] + (N,))
